# baseline (speedup 1.0000x reference)
import functools

import jax
import jax.numpy as jnp
from jax import lax
from jax.experimental import pallas as pl
from jax.experimental.pallas import tpu as pltpu
from jax.experimental.pallas import tpu_sc as plsc

_NC = 2
_NS = 16
_NW = _NC * _NS

_L = 128
_NSTREAM = 16
_CHUNK = _L * _NSTREAM


def _gather_kernel(J, I, D, table_hbm, idxT_hbm, out_hbm,
                   idx_v, rows_v, sem_i0, sem_i1, sem_g, sem_s0, sem_s1):
    e_per_j = I // _CHUNK
    n_units = (J * I) // (_CHUNK * _NW)
    n_pairs = n_units // 2
    wid = lax.axis_index("s") * _NC + lax.axis_index("c")
    u0 = wid * n_units

    def idx_load(u_glob, buf, sem):
        j = u_glob // e_per_j
        e = u_glob % e_per_j
        pltpu.async_copy(
            idxT_hbm.at[j, pl.ds(e * _CHUNK, _CHUNK)], idx_v.at[buf], sem)

    idx_load(u0, 0, sem_i0)

    def pair_body(p, carry):
        for b in (0, 1):
            u = 2 * p + b
            u_glob = u0 + u
            j = u_glob // e_per_j
            e = u_glob % e_per_j
            sem_i = (sem_i0, sem_i1)[b]
            sem_s = (sem_s0, sem_s1)[b]

            @pl.when(p >= 1)
            def _():
                for d in range(D):
                    pltpu.make_async_copy(
                        rows_v.at[b, :, :, d],
                        out_hbm.at[0, pl.ds(0, _NSTREAM), d % 8, :],
                        sem_s,
                    ).wait()

            pltpu.make_async_copy(
                idxT_hbm.at[0, pl.ds(0, _CHUNK)], idx_v.at[b], sem_i
            ).wait()

            def _prefetch():
                idx_load(u_glob + 1, 1 - b, (sem_i1, sem_i0)[b])
            if b == 0:
                _prefetch()
            else:
                pl.when(p < n_pairs - 1)(_prefetch)

            copies = []
            for k in range(_NSTREAM):
                cp = pltpu.async_copy(
                    table_hbm.at[idx_v.at[b, pl.ds(k * _L, _L)]],
                    rows_v.at[b, k],
                    sem_g,
                )
                copies.append(cp)
            for cp in copies:
                cp.wait()

            for d in range(D):
                pltpu.async_copy(
                    rows_v.at[b, :, :, d],
                    out_hbm.at[j * 2 + d // 8, pl.ds(e * _NSTREAM, _NSTREAM),
                               d % 8, :],
                    sem_s,
                )
        return carry

    lax.fori_loop(0, n_pairs, pair_body, 0)

    for b in (0, 1):
        for d in range(D):
            pltpu.make_async_copy(
                rows_v.at[b, :, :, d],
                out_hbm.at[0, pl.ds(0, _NSTREAM), d % 8, :],
                (sem_s0, sem_s1)[b],
            ).wait()


def kernel(emb, idx):
    V, D = emb.shape
    B0, B1 = idx.shape
    I, J = B0, B1
    idxT = idx.T

    mesh = plsc.VectorSubcoreMesh(core_axis_name="c", subcore_axis_name="s")
    gather = functools.partial(
        pl.kernel,
        mesh=mesh,
        out_type=jax.ShapeDtypeStruct((J * 2, I // _L, 8, _L), jnp.float32),
        scratch_types=[
            pltpu.VMEM((2, _CHUNK), jnp.int32),
            pltpu.VMEM((2, _NSTREAM, _L, D), jnp.float32),
            pltpu.SemaphoreType.DMA,
            pltpu.SemaphoreType.DMA,
            pltpu.SemaphoreType.DMA,
            pltpu.SemaphoreType.DMA,
            pltpu.SemaphoreType.DMA,
        ],
        compiler_params=pltpu.CompilerParams(use_tc_tiling_on_sc=False),
    )(functools.partial(_gather_kernel, J, I, D))

    k4 = gather(emb, idxT)
    out = (k4.reshape(J, 2, I // _L, 8, _L)
             .transpose(2, 4, 0, 1, 3)
             .reshape(I, J, D))
    return out


# speedup vs baseline: 85.0533x; 85.0533x over previous
import functools

import jax
import jax.numpy as jnp
from jax import lax
from jax.experimental import pallas as pl
from jax.experimental.pallas import tpu as pltpu
from jax.experimental.pallas import tpu_sc as plsc

_NC = 2
_NS = 16
_NW = _NC * _NS

_L = 128
_NSTREAM = 8
_CHUNK = _L * _NSTREAM
_D = 16


def _store_unit(out_hbm, rowsT_v, sem, bb, j, e):
    for d in range(_D):
        pltpu.async_copy(
            rowsT_v.at[bb, d],
            out_hbm.at[j * 2 + d // 8, pl.ds(e * _NSTREAM, _NSTREAM),
                       d % 8, :],
            sem,
        )


def _drain_stores(out_hbm, rowsT_v, sem, bb):
    for d in range(_D):
        pltpu.make_async_copy(
            rowsT_v.at[bb, d],
            out_hbm.at[0, pl.ds(0, _NSTREAM), d % 8, :],
            sem,
        ).wait()


def _transpose_unit(rows_v, rowsT_v, bb):
    iota = lax.iota(jnp.int32, 16)
    lvecs = [iota + (16 * g) for g in range(_L // 16)]
    dvecs = [jnp.full((16,), d, jnp.int32) for d in range(_D)]

    def body(k, carry):
        kvec = jnp.full((16,), 0, jnp.int32) + k
        for d in range(_D):
            for g in range(_L // 16):
                vec = plsc.load_gather(rows_v.at[bb], [kvec, lvecs[g], dvecs[d]])
                rowsT_v[bb, d, k, pl.ds(g * 16, 16)] = vec
        return carry

    lax.fori_loop(0, _NSTREAM, body, 0)


def _gather_kernel(J, I, table_hbm, idxT_hbm, out_hbm,
                   idx_v, rows_v, rowsT_v,
                   sem_i0, sem_i1, sem_g0, sem_g1, sem_s0, sem_s1):
    e_per_j = I // _CHUNK
    n_units = (J * I) // (_CHUNK * _NW)
    n_pairs = n_units // 2
    wid = lax.axis_index("s") * _NC + lax.axis_index("c")
    u0 = wid * n_units

    def idx_load(u_glob, buf, sem):
        j = u_glob // e_per_j
        e = u_glob % e_per_j
        pltpu.async_copy(
            idxT_hbm.at[j, pl.ds(e * _CHUNK, _CHUNK)], idx_v.at[buf], sem)

    def fire_gathers(bb, sem):
        for k in range(_NSTREAM):
            pltpu.async_copy(
                table_hbm.at[idx_v.at[bb, pl.ds(k * _L, _L)]],
                rows_v.at[bb, k],
                sem,
            )

    def drain_gathers(bb, sem):
        for k in range(_NSTREAM):
            pltpu.make_async_copy(
                table_hbm.at[idx_v.at[bb, pl.ds(0, _L)]],
                rows_v.at[bb, k],
                sem,
            ).wait()

    idx_load(u0, 0, sem_i0)

    def pair_body(p, carry):
        for b in (0, 1):
            u = 2 * p + b
            u_glob = u0 + u
            sem_i = (sem_i0, sem_i1)[b]
            sem_g = (sem_g0, sem_g1)[b]

            pltpu.make_async_copy(
                idxT_hbm.at[0, pl.ds(0, _CHUNK)], idx_v.at[b], sem_i
            ).wait()

            fire_gathers(b, sem_g)

            def middle(prefetch=True):
                drain_gathers(1 - b, (sem_g1, sem_g0)[b])
                if prefetch:
                    idx_load(u_glob + 1, 1 - b, (sem_i1, sem_i0)[b])

            def tail():
                v_glob = u_glob - 1
                jv = v_glob // e_per_j
                ev = v_glob % e_per_j
                _transpose_unit(rows_v, rowsT_v, 1 - b)
                _store_unit(out_hbm, rowsT_v, (sem_s1, sem_s0)[b],
                            1 - b, jv, ev)

            if b == 0:
                @pl.when(p >= 1)
                def _():
                    middle()

                @pl.when(p >= 2)
                def _():
                    _drain_stores(out_hbm, rowsT_v, sem_s1, 1)

                @pl.when(p >= 1)
                def _():
                    tail()

                @pl.when(p == 0)
                def _():
                    idx_load(u_glob + 1, 1, sem_i1)
            else:
                @pl.when(p < n_pairs - 1)
                def _():
                    middle()

                @pl.when(p == n_pairs - 1)
                def _():
                    middle(prefetch=False)

                @pl.when(p >= 1)
                def _():
                    _drain_stores(out_hbm, rowsT_v, sem_s0, 0)

                tail()
        return carry

    lax.fori_loop(0, n_pairs, pair_body, 0)

    u_last = u0 + n_units - 1
    drain_gathers(1, sem_g1)
    _drain_stores(out_hbm, rowsT_v, sem_s1, 1)
    _transpose_unit(rows_v, rowsT_v, 1)
    _store_unit(out_hbm, rowsT_v, sem_s1, 1,
                u_last // e_per_j, u_last % e_per_j)
    _drain_stores(out_hbm, rowsT_v, sem_s0, 0)
    _drain_stores(out_hbm, rowsT_v, sem_s1, 1)


def kernel(emb, idx):
    V, D = emb.shape
    B0, B1 = idx.shape
    I, J = B0, B1
    idxT = idx.T

    mesh = plsc.VectorSubcoreMesh(core_axis_name="c", subcore_axis_name="s")
    gather = functools.partial(
        pl.kernel,
        mesh=mesh,
        out_type=jax.ShapeDtypeStruct((J * 2, I // _L, 8, _L), jnp.float32),
        scratch_types=[
            pltpu.VMEM((2, _CHUNK), jnp.int32),
            pltpu.VMEM((2, _NSTREAM, _L, _D), jnp.float32),
            pltpu.VMEM((2, _D, _NSTREAM, _L), jnp.float32),
            pltpu.SemaphoreType.DMA,
            pltpu.SemaphoreType.DMA,
            pltpu.SemaphoreType.DMA,
            pltpu.SemaphoreType.DMA,
            pltpu.SemaphoreType.DMA,
            pltpu.SemaphoreType.DMA,
        ],
        compiler_params=pltpu.CompilerParams(
            use_tc_tiling_on_sc=False, needs_layout_passes=False),
    )(functools.partial(_gather_kernel, J, I))

    k4 = gather(emb, idxT)
    out = (k4.reshape(J, 2, I // _L, 8, _L)
             .transpose(2, 4, 0, 1, 3)
             .reshape(I, J, D))
    return out


# speedup vs baseline: 115.3515x; 1.3562x over previous
import functools

import jax
import jax.numpy as jnp
from jax import lax
from jax.experimental import pallas as pl
from jax.experimental.pallas import tpu as pltpu
from jax.experimental.pallas import tpu_sc as plsc

_NC = 2
_NS = 16
_NW = _NC * _NS

_L = 128
_NSTREAM = 8
_CHUNK = _L * _NSTREAM
_D = 16


def _store_unit(out_hbm, rowsT_v, sem, bb, j, e):
    for d in range(_D):
        pltpu.async_copy(
            rowsT_v.at[bb, d],
            out_hbm.at[j * 2 + d // 8, pl.ds(e * _NSTREAM, _NSTREAM),
                       d % 8, :],
            sem,
        )


def _drain_stores(out_hbm, rowsT_v, sem, bb):
    for d in range(_D):
        pltpu.make_async_copy(
            rowsT_v.at[bb, d],
            out_hbm.at[0, pl.ds(0, _NSTREAM), d % 8, :],
            sem,
        ).wait()


def _transpose_unit(rows_v, rowsT_v, bb):
    iota = lax.iota(jnp.int32, 16)
    lvecs = [iota + (16 * g) for g in range(_L // 16)]
    dvecs = [jnp.full((16,), d, jnp.int32) for d in range(_D)]

    def body(k, carry):
        kvec = jnp.full((16,), 0, jnp.int32) + k
        for d in range(_D):
            vecs = [
                plsc.load_gather(rows_v.at[bb], [kvec, lvecs[g], dvecs[d]])
                for g in range(_L // 16)
            ]
            for g in range(_L // 16):
                rowsT_v[bb, d, k, pl.ds(g * 16, 16)] = vecs[g]
        return carry

    lax.fori_loop(0, _NSTREAM, body, 0)


def _gather_kernel(J, I, table_hbm, idxT_hbm, out_hbm,
                   idx_v, rows_v, rowsT_v,
                   sem_i0, sem_i1, sem_g0, sem_g1, sem_s0, sem_s1):
    e_per_j = I // _CHUNK
    n_units = (J * I) // (_CHUNK * _NW)
    n_pairs = n_units // 2
    wid = lax.axis_index("s") * _NC + lax.axis_index("c")
    u0 = wid * n_units

    def idx_load(u_glob, buf, sem):
        j = u_glob // e_per_j
        e = u_glob % e_per_j
        pltpu.async_copy(
            idxT_hbm.at[j, pl.ds(e * _CHUNK, _CHUNK)], idx_v.at[buf], sem)

    def fire_gathers(bb, sem):
        for k in range(_NSTREAM):
            pltpu.async_copy(
                table_hbm.at[idx_v.at[bb, pl.ds(k * _L, _L)]],
                rows_v.at[bb, k],
                sem,
            )

    def drain_gathers(bb, sem):
        for k in range(_NSTREAM):
            pltpu.make_async_copy(
                table_hbm.at[idx_v.at[bb, pl.ds(0, _L)]],
                rows_v.at[bb, k],
                sem,
            ).wait()

    idx_load(u0, 0, sem_i0)

    def pair_body(p, carry):
        for b in (0, 1):
            u = 2 * p + b
            u_glob = u0 + u
            sem_i = (sem_i0, sem_i1)[b]
            sem_g = (sem_g0, sem_g1)[b]

            pltpu.make_async_copy(
                idxT_hbm.at[0, pl.ds(0, _CHUNK)], idx_v.at[b], sem_i
            ).wait()

            fire_gathers(b, sem_g)

            def middle(prefetch=True):
                drain_gathers(1 - b, (sem_g1, sem_g0)[b])
                if prefetch:
                    idx_load(u_glob + 1, 1 - b, (sem_i1, sem_i0)[b])

            def tail():
                v_glob = u_glob - 1
                jv = v_glob // e_per_j
                ev = v_glob % e_per_j
                _transpose_unit(rows_v, rowsT_v, 1 - b)
                _store_unit(out_hbm, rowsT_v, (sem_s1, sem_s0)[b],
                            1 - b, jv, ev)

            if b == 0:
                @pl.when(p >= 1)
                def _():
                    middle()

                @pl.when(p >= 2)
                def _():
                    _drain_stores(out_hbm, rowsT_v, sem_s1, 1)

                @pl.when(p >= 1)
                def _():
                    tail()

                @pl.when(p == 0)
                def _():
                    idx_load(u_glob + 1, 1, sem_i1)
            else:
                @pl.when(p < n_pairs - 1)
                def _():
                    middle()

                @pl.when(p == n_pairs - 1)
                def _():
                    middle(prefetch=False)

                @pl.when(p >= 1)
                def _():
                    _drain_stores(out_hbm, rowsT_v, sem_s0, 0)

                tail()
        return carry

    lax.fori_loop(0, n_pairs, pair_body, 0)

    u_last = u0 + n_units - 1
    drain_gathers(1, sem_g1)
    _drain_stores(out_hbm, rowsT_v, sem_s1, 1)
    _transpose_unit(rows_v, rowsT_v, 1)
    _store_unit(out_hbm, rowsT_v, sem_s1, 1,
                u_last // e_per_j, u_last % e_per_j)
    _drain_stores(out_hbm, rowsT_v, sem_s0, 0)
    _drain_stores(out_hbm, rowsT_v, sem_s1, 1)


def kernel(emb, idx):
    V, D = emb.shape
    B0, B1 = idx.shape
    I, J = B0, B1
    idxT = idx.T

    mesh = plsc.VectorSubcoreMesh(core_axis_name="c", subcore_axis_name="s")
    gather = functools.partial(
        pl.kernel,
        mesh=mesh,
        out_type=jax.ShapeDtypeStruct((J * 2, I // _L, 8, _L), jnp.float32),
        scratch_types=[
            pltpu.VMEM((2, _CHUNK), jnp.int32),
            pltpu.VMEM((2, _NSTREAM, _L, _D), jnp.float32),
            pltpu.VMEM((2, _D, _NSTREAM, _L), jnp.float32),
            pltpu.SemaphoreType.DMA,
            pltpu.SemaphoreType.DMA,
            pltpu.SemaphoreType.DMA,
            pltpu.SemaphoreType.DMA,
            pltpu.SemaphoreType.DMA,
            pltpu.SemaphoreType.DMA,
        ],
        compiler_params=pltpu.CompilerParams(
            use_tc_tiling_on_sc=False, needs_layout_passes=False),
    )(functools.partial(_gather_kernel, J, I))

    k4 = gather(emb, idxT)
    out = (k4.reshape(J, 2, I // _L, 8, _L)
             .transpose(2, 4, 0, 1, 3)
             .reshape(I, J, D))
    return out


# speedup vs baseline: 122.8663x; 1.0651x over previous
import functools

import jax
import jax.numpy as jnp
from jax import lax
from jax.experimental import pallas as pl
from jax.experimental.pallas import tpu as pltpu
from jax.experimental.pallas import tpu_sc as plsc

_NC = 2
_NS = 16
_NW = _NC * _NS

_L = 128
_NSTREAM = 8
_CHUNK = _L * _NSTREAM
_D = 16


def _store_unit(out_hbm, rowsT_v, sem, bb, j, e):
    for d in range(_D):
        pltpu.async_copy(
            rowsT_v.at[bb, d],
            out_hbm.at[j * 2 + d // 8, pl.ds(e * _NSTREAM, _NSTREAM),
                       d % 8, :],
            sem,
        )


def _drain_stores(out_hbm, rowsT_v, sem, bb):
    for d in range(_D):
        pltpu.make_async_copy(
            rowsT_v.at[bb, d],
            out_hbm.at[0, pl.ds(0, _NSTREAM), d % 8, :],
            sem,
        ).wait()


def _transpose_unit(rows_v, rowsT_v, bb):
    iota = lax.iota(jnp.int32, 16)
    lvecs = [iota + (16 * g) for g in range(_L // 16)]
    dvecs = [jnp.full((16,), d, jnp.int32) for d in range(_D)]

    @plsc.parallel_loop(0, _NSTREAM, unroll=2)
    def _(k):
        kvec = jnp.full((16,), 0, jnp.int32) + k
        for d in range(_D):
            vecs = [
                plsc.load_gather(rows_v.at[bb], [kvec, lvecs[g], dvecs[d]])
                for g in range(_L // 16)
            ]
            for g in range(_L // 16):
                rowsT_v[bb, d, k, pl.ds(g * 16, 16)] = vecs[g]


def _gather_kernel(J, I, table_hbm, idxT_hbm, out_hbm,
                   idx_v, rows_v, rowsT_v,
                   sem_i0, sem_i1, sem_g0, sem_g1, sem_s0, sem_s1):
    e_per_j = I // _CHUNK
    n_units = (J * I) // (_CHUNK * _NW)
    n_pairs = n_units // 2
    wid = lax.axis_index("s") * _NC + lax.axis_index("c")
    u0 = wid * n_units

    def idx_load(u_glob, buf, sem):
        j = u_glob // e_per_j
        e = u_glob % e_per_j
        pltpu.async_copy(
            idxT_hbm.at[j, pl.ds(e * _CHUNK, _CHUNK)], idx_v.at[buf], sem)

    def fire_gathers(bb, sem):
        for k in range(_NSTREAM):
            pltpu.async_copy(
                table_hbm.at[idx_v.at[bb, pl.ds(k * _L, _L)]],
                rows_v.at[bb, k],
                sem,
            )

    def drain_gathers(bb, sem):
        for k in range(_NSTREAM):
            pltpu.make_async_copy(
                table_hbm.at[idx_v.at[bb, pl.ds(0, _L)]],
                rows_v.at[bb, k],
                sem,
            ).wait()

    idx_load(u0, 0, sem_i0)

    def pair_body(p, carry):
        for b in (0, 1):
            u = 2 * p + b
            u_glob = u0 + u
            sem_i = (sem_i0, sem_i1)[b]
            sem_g = (sem_g0, sem_g1)[b]

            pltpu.make_async_copy(
                idxT_hbm.at[0, pl.ds(0, _CHUNK)], idx_v.at[b], sem_i
            ).wait()

            fire_gathers(b, sem_g)

            def middle(prefetch=True):
                drain_gathers(1 - b, (sem_g1, sem_g0)[b])
                if prefetch:
                    idx_load(u_glob + 1, 1 - b, (sem_i1, sem_i0)[b])

            def tail():
                v_glob = u_glob - 1
                jv = v_glob // e_per_j
                ev = v_glob % e_per_j
                _transpose_unit(rows_v, rowsT_v, 1 - b)
                _store_unit(out_hbm, rowsT_v, (sem_s1, sem_s0)[b],
                            1 - b, jv, ev)

            if b == 0:
                @pl.when(p >= 1)
                def _():
                    middle()

                @pl.when(p >= 2)
                def _():
                    _drain_stores(out_hbm, rowsT_v, sem_s1, 1)

                @pl.when(p >= 1)
                def _():
                    tail()

                @pl.when(p == 0)
                def _():
                    idx_load(u_glob + 1, 1, sem_i1)
            else:
                @pl.when(p < n_pairs - 1)
                def _():
                    middle()

                @pl.when(p == n_pairs - 1)
                def _():
                    middle(prefetch=False)

                @pl.when(p >= 1)
                def _():
                    _drain_stores(out_hbm, rowsT_v, sem_s0, 0)

                tail()
        return carry

    lax.fori_loop(0, n_pairs, pair_body, 0)

    u_last = u0 + n_units - 1
    drain_gathers(1, sem_g1)
    _drain_stores(out_hbm, rowsT_v, sem_s1, 1)
    _transpose_unit(rows_v, rowsT_v, 1)
    _store_unit(out_hbm, rowsT_v, sem_s1, 1,
                u_last // e_per_j, u_last % e_per_j)
    _drain_stores(out_hbm, rowsT_v, sem_s0, 0)
    _drain_stores(out_hbm, rowsT_v, sem_s1, 1)


def kernel(emb, idx):
    V, D = emb.shape
    B0, B1 = idx.shape
    I, J = B0, B1
    idxT = idx.T

    mesh = plsc.VectorSubcoreMesh(core_axis_name="c", subcore_axis_name="s")
    gather = functools.partial(
        pl.kernel,
        mesh=mesh,
        out_type=jax.ShapeDtypeStruct((J * 2, I // _L, 8, _L), jnp.float32),
        scratch_types=[
            pltpu.VMEM((2, _CHUNK), jnp.int32),
            pltpu.VMEM((2, _NSTREAM, _L, _D), jnp.float32),
            pltpu.VMEM((2, _D, _NSTREAM, _L), jnp.float32),
            pltpu.SemaphoreType.DMA,
            pltpu.SemaphoreType.DMA,
            pltpu.SemaphoreType.DMA,
            pltpu.SemaphoreType.DMA,
            pltpu.SemaphoreType.DMA,
            pltpu.SemaphoreType.DMA,
        ],
        compiler_params=pltpu.CompilerParams(
            use_tc_tiling_on_sc=False, needs_layout_passes=False),
    )(functools.partial(_gather_kernel, J, I))

    k4 = gather(emb, idxT)
    out = (k4.reshape(J, 2, I // _L, 8, _L)
             .transpose(2, 4, 0, 1, 3)
             .reshape(I, J, D))
    return out


# speedup vs baseline: 123.6088x; 1.0060x over previous
import functools

import jax
import jax.numpy as jnp
from jax import lax
from jax.experimental import pallas as pl
from jax.experimental.pallas import tpu as pltpu
from jax.experimental.pallas import tpu_sc as plsc

_NC = 2
_NS = 16
_NW = _NC * _NS

_L = 128
_NSTREAM = 8
_CHUNK = _L * _NSTREAM
_D = 16


def _store_unit(out_hbm, rowsT_v, sem, bb, j, e):
    for d in range(_D):
        pltpu.async_copy(
            rowsT_v.at[bb, d],
            out_hbm.at[j * 2 + d // 8, pl.ds(e * _NSTREAM, _NSTREAM),
                       d % 8, :],
            sem,
        )


def _drain_stores(out_hbm, rowsT_v, sem, bb):
    for d in range(_D):
        pltpu.make_async_copy(
            rowsT_v.at[bb, d],
            out_hbm.at[0, pl.ds(0, _NSTREAM), d % 8, :],
            sem,
        ).wait()


def _transpose_unit(rows_v, rowsT_v, bb):
    iota = lax.iota(jnp.int32, 16)
    lvecs = [iota + (16 * g) for g in range(_L // 16)]
    dvecs = [jnp.full((16,), d, jnp.int32) for d in range(_D)]

    @plsc.parallel_loop(0, _NSTREAM, unroll=2)
    def _(k):
        kvec = jnp.full((16,), 0, jnp.int32) + k
        for d in range(_D):
            vecs = [
                plsc.load_gather(rows_v.at[bb], [kvec, lvecs[g], dvecs[d]])
                for g in range(_L // 16)
            ]
            for g in range(_L // 16):
                rowsT_v[bb, d, k, pl.ds(g * 16, 16)] = vecs[g]


def _gather_kernel(J, I, table_hbm, idxT_hbm, out_hbm,
                   idx_v, rows_v, rowsT_v,
                   sem_i0, sem_i1, sem_g0, sem_g1, sem_s0, sem_s1):
    e_per_j = I // _CHUNK
    n_units = (J * I) // (_CHUNK * _NW)
    n_pairs = n_units // 2
    wid = lax.axis_index("s") * _NC + lax.axis_index("c")
    u0 = wid * n_units

    def idx_load(u_glob, buf, sem):
        j = u_glob // e_per_j
        e = u_glob % e_per_j
        pltpu.async_copy(
            idxT_hbm.at[j // 8, pl.ds(e * _NSTREAM, _NSTREAM), j % 8, :],
            idx_v.at[buf], sem)

    def fire_gathers(bb, sem):
        for k in range(_NSTREAM):
            pltpu.async_copy(
                table_hbm.at[idx_v.at[bb, k]],
                rows_v.at[bb, k],
                sem,
            )

    def drain_gathers(bb, sem):
        for k in range(_NSTREAM):
            pltpu.make_async_copy(
                table_hbm.at[idx_v.at[bb, 0]],
                rows_v.at[bb, k],
                sem,
            ).wait()

    idx_load(u0, 0, sem_i0)

    def pair_body(p, carry):
        for b in (0, 1):
            u = 2 * p + b
            u_glob = u0 + u
            sem_i = (sem_i0, sem_i1)[b]
            sem_g = (sem_g0, sem_g1)[b]

            pltpu.make_async_copy(
                idxT_hbm.at[0, pl.ds(0, _NSTREAM), 0, :], idx_v.at[b], sem_i
            ).wait()

            fire_gathers(b, sem_g)

            def middle(prefetch=True):
                drain_gathers(1 - b, (sem_g1, sem_g0)[b])
                if prefetch:
                    idx_load(u_glob + 1, 1 - b, (sem_i1, sem_i0)[b])

            def tail():
                v_glob = u_glob - 1
                jv = v_glob // e_per_j
                ev = v_glob % e_per_j
                _transpose_unit(rows_v, rowsT_v, 1 - b)
                _store_unit(out_hbm, rowsT_v, (sem_s1, sem_s0)[b],
                            1 - b, jv, ev)

            if b == 0:
                @pl.when(p >= 1)
                def _():
                    middle()

                @pl.when(p >= 2)
                def _():
                    _drain_stores(out_hbm, rowsT_v, sem_s1, 1)

                @pl.when(p >= 1)
                def _():
                    tail()

                @pl.when(p == 0)
                def _():
                    idx_load(u_glob + 1, 1, sem_i1)
            else:
                @pl.when(p < n_pairs - 1)
                def _():
                    middle()

                @pl.when(p == n_pairs - 1)
                def _():
                    middle(prefetch=False)

                @pl.when(p >= 1)
                def _():
                    _drain_stores(out_hbm, rowsT_v, sem_s0, 0)

                tail()
        return carry

    lax.fori_loop(0, n_pairs, pair_body, 0)

    u_last = u0 + n_units - 1
    drain_gathers(1, sem_g1)
    _drain_stores(out_hbm, rowsT_v, sem_s1, 1)
    _transpose_unit(rows_v, rowsT_v, 1)
    _store_unit(out_hbm, rowsT_v, sem_s1, 1,
                u_last // e_per_j, u_last % e_per_j)
    _drain_stores(out_hbm, rowsT_v, sem_s0, 0)
    _drain_stores(out_hbm, rowsT_v, sem_s1, 1)


def kernel(emb, idx):
    V, D = emb.shape
    B0, B1 = idx.shape
    I, J = B0, B1
    idxT = idx.reshape(I // _L, _L, J // 8, 8).transpose(2, 0, 3, 1)

    mesh = plsc.VectorSubcoreMesh(core_axis_name="c", subcore_axis_name="s")
    gather = functools.partial(
        pl.kernel,
        mesh=mesh,
        out_type=jax.ShapeDtypeStruct((J * 2, I // _L, 8, _L), jnp.float32),
        scratch_types=[
            pltpu.VMEM((2, _NSTREAM, _L), jnp.int32),
            pltpu.VMEM((2, _NSTREAM, _L, _D), jnp.float32),
            pltpu.VMEM((2, _D, _NSTREAM, _L), jnp.float32),
            pltpu.SemaphoreType.DMA,
            pltpu.SemaphoreType.DMA,
            pltpu.SemaphoreType.DMA,
            pltpu.SemaphoreType.DMA,
            pltpu.SemaphoreType.DMA,
            pltpu.SemaphoreType.DMA,
        ],
        compiler_params=pltpu.CompilerParams(
            use_tc_tiling_on_sc=False, needs_layout_passes=False),
    )(functools.partial(_gather_kernel, J, I))

    k4 = gather(emb, idxT)
    out = (k4.reshape(J, 2, I // _L, 8, _L)
             .transpose(2, 4, 0, 1, 3)
             .reshape(I, J, D))
    return out
